# baseline (speedup 1.0000x reference)
import functools

import jax
import jax.numpy as jnp
from jax import lax
from jax.experimental import pallas as pl
from jax.experimental.pallas import tpu as pltpu
from jax.experimental.pallas import tpu_sc as plsc

NUM_POSES = 100000
POSE_DIM = 9
BATCH = 16384

_NUM_CORES = 2
_NUM_SUBCORES = 16
_NUM_WORKERS = _NUM_CORES * _NUM_SUBCORES
_B_PER_W = BATCH // _NUM_WORKERS
_COL_BLKS = NUM_POSES // 8
_LANES = 16

_mesh = plsc.VectorSubcoreMesh(core_axis_name="c", subcore_axis_name="s")

_sc_params = pltpu.CompilerParams(use_tc_tiling_on_sc=False,
                                  needs_layout_passes=False,
                                  skip_device_barrier=True,
                                  disable_bounds_checks=True,
                                  disable_semaphore_checks=True)


@functools.partial(
    pl.kernel,
    mesh=_mesh,
    out_type=(jax.ShapeDtypeStruct((BATCH,), jnp.int32),
              jax.ShapeDtypeStruct((BATCH,), jnp.int32)),
    scratch_types=[
        pltpu.VMEM((_B_PER_W,), jnp.int32),
        pltpu.VMEM((_B_PER_W,), jnp.int32),
        pltpu.VMEM((_B_PER_W,), jnp.int32),
    ],
    compiler_params=_sc_params,
)
def _prep_sc(idx_hbm, blk_hbm, low_hbm, idx_v, blk_v, low_v):
    wid = lax.axis_index("s") * _NUM_CORES + lax.axis_index("c")
    base = wid * _B_PER_W
    pltpu.sync_copy(idx_hbm.at[pl.ds(base, _B_PER_W)], idx_v)

    def prep_body(t, carry):
        for u in range(4):
            s = (t * 4 + u) * _LANES
            v = idx_v[pl.ds(s, _LANES)]
            low_v[pl.ds(s, _LANES)] = lax.bitwise_and(v, 7)
            blk_v[pl.ds(s, _LANES)] = lax.shift_right_logical(v, 3)
        return carry

    lax.fori_loop(0, _B_PER_W // _LANES // 4, prep_body, 0)
    pltpu.sync_copy(blk_v, blk_hbm.at[pl.ds(base, _B_PER_W)])
    pltpu.sync_copy(low_v, low_hbm.at[pl.ds(base, _B_PER_W)])


@functools.partial(
    pl.kernel,
    mesh=_mesh,
    out_type=jax.ShapeDtypeStruct((POSE_DIM, BATCH), jnp.float32),
    scratch_types=[
        pltpu.VMEM((_B_PER_W,), jnp.int32),
        pltpu.VMEM((_B_PER_W,), jnp.int32),
        pltpu.VMEM((POSE_DIM * _B_PER_W, 8), jnp.float32),
        pltpu.VMEM((POSE_DIM, _B_PER_W), jnp.float32),
    ] + [pltpu.SemaphoreType.DMA] * POSE_DIM,
    compiler_params=_sc_params,
)
def _gather_sc(tab_hbm, blk_hbm, low_hbm, out_hbm, blk_v, low_v, win_v,
               rows_v, *sems):
    wid = lax.axis_index("s") * _NUM_CORES + lax.axis_index("c")
    base = wid * _B_PER_W
    pltpu.sync_copy(blk_hbm.at[pl.ds(base, _B_PER_W)], blk_v)
    pltpu.sync_copy(low_hbm.at[pl.ds(base, _B_PER_W)], low_v)

    iota = lax.iota(jnp.int32, _LANES)

    copies = [
        pltpu.async_copy(
            tab_hbm.at[pl.ds(j * _COL_BLKS, _COL_BLKS)].at[blk_v],
            win_v.at[pl.ds(j * _B_PER_W, _B_PER_W)], sems[j])
        for j in range(POSE_DIM)
    ]

    out_copies = []
    for j in range(POSE_DIM):
        copies[j].wait()

        def col_body(t, carry, j=j):
            for u in range(4):
                s = (t * 4 + u) * _LANES
                lo = low_v[pl.ds(s, _LANES)]
                rows16 = (j * _B_PER_W + s) + iota
                rows_v[j, pl.ds(s, _LANES)] = (
                    plsc.load_gather(win_v, [rows16, lo]))
            return carry

        lax.fori_loop(0, _B_PER_W // _LANES // 4, col_body, 0)
        out_copies.append(pltpu.async_copy(
            rows_v.at[j], out_hbm.at[j, pl.ds(base, _B_PER_W)], sems[j]))

    for cp in out_copies:
        cp.wait()


def kernel(d9, i):
    tab = d9.T.reshape(NUM_POSES * POSE_DIM // 8, 8)
    blk, low = _prep_sc(i.astype(jnp.int32))
    out = _gather_sc(tab, blk, low)
    return out.T


# speedup vs baseline: 1.0767x; 1.0767x over previous
import functools

import jax
import jax.numpy as jnp
from jax import lax
from jax.experimental import pallas as pl
from jax.experimental.pallas import tpu as pltpu
from jax.experimental.pallas import tpu_sc as plsc

NUM_POSES = 100000
POSE_DIM = 9
BATCH = 16384

_NUM_CORES = 2
_NUM_SUBCORES = 16
_NUM_WORKERS = _NUM_CORES * _NUM_SUBCORES
_B_PER_W = BATCH // _NUM_WORKERS
_COL_BLKS = NUM_POSES // 8
_LANES = 16

_mesh = plsc.VectorSubcoreMesh(core_axis_name="c", subcore_axis_name="s")


@functools.partial(
    pl.kernel,
    mesh=_mesh,
    out_type=jax.ShapeDtypeStruct((POSE_DIM, BATCH), jnp.float32),
    scratch_types=[
        pltpu.VMEM((_B_PER_W,), jnp.int32),
        pltpu.VMEM((_B_PER_W,), jnp.int32),
        pltpu.VMEM((_B_PER_W,), jnp.int32),
        pltpu.VMEM((POSE_DIM * _B_PER_W, 8), jnp.float32),
        pltpu.VMEM((POSE_DIM, _B_PER_W), jnp.float32),
    ] + [pltpu.SemaphoreType.DMA] * POSE_DIM,
    compiler_params=pltpu.CompilerParams(use_tc_tiling_on_sc=False,
                                         needs_layout_passes=False,
                                         skip_device_barrier=True,
                                         disable_bounds_checks=True,
                                         disable_semaphore_checks=True),
)
def _gather_sc(tab_hbm, idx_hbm, out_hbm, idx_v, low_v, blk_v, win_v,
               rows_v, *sems):
    wid = lax.axis_index("s") * _NUM_CORES + lax.axis_index("c")
    base = wid * _B_PER_W
    pltpu.sync_copy(idx_hbm.at[pl.ds(base, _B_PER_W)], idx_v)

    iota = lax.iota(jnp.int32, _LANES)

    def prep_body(t, carry):
        for u in range(4):
            s = (t * 4 + u) * _LANES
            v = idx_v[pl.ds(s, _LANES)]
            low_v[pl.ds(s, _LANES)] = lax.bitwise_and(v, 7)
            blk_v[pl.ds(s, _LANES)] = lax.shift_right_logical(v, 3)
        return carry

    lax.fori_loop(0, _B_PER_W // _LANES // 4, prep_body, 0)

    copies = [
        pltpu.async_copy(
            tab_hbm.at[pl.ds(j * _COL_BLKS, _COL_BLKS)].at[blk_v],
            win_v.at[pl.ds(j * _B_PER_W, _B_PER_W)], sems[j])
        for j in range(POSE_DIM)
    ]

    out_copies = []
    for j in range(POSE_DIM):
        copies[j].wait()

        def col_body(t, carry, j=j):
            for u in range(4):
                s = (t * 4 + u) * _LANES
                lo = low_v[pl.ds(s, _LANES)]
                rows16 = (j * _B_PER_W + s) + iota
                rows_v[j, pl.ds(s, _LANES)] = (
                    plsc.load_gather(win_v, [rows16, lo]))
            return carry

        lax.fori_loop(0, _B_PER_W // _LANES // 4, col_body, 0)
        out_copies.append(pltpu.async_copy(
            rows_v.at[j], out_hbm.at[j, pl.ds(base, _B_PER_W)], sems[j]))

    for cp in out_copies:
        cp.wait()


def kernel(d9, i):
    tab = d9.T.reshape(NUM_POSES * POSE_DIM // 8, 8)
    out = _gather_sc(tab, i.astype(jnp.int32))
    return out.T


# speedup vs baseline: 1.0851x; 1.0078x over previous
import functools

import jax
import jax.numpy as jnp
from jax import lax
from jax.experimental import pallas as pl
from jax.experimental.pallas import tpu as pltpu
from jax.experimental.pallas import tpu_sc as plsc

NUM_POSES = 100000
POSE_DIM = 9
BATCH = 16384

_NUM_CORES = 2
_NUM_SUBCORES = 16
_NUM_WORKERS = _NUM_CORES * _NUM_SUBCORES
_B_PER_W = BATCH // _NUM_WORKERS
_COL_BLKS = NUM_POSES // 8
_LANES = 16

_mesh = plsc.VectorSubcoreMesh(core_axis_name="c", subcore_axis_name="s")


@functools.partial(
    pl.kernel,
    mesh=_mesh,
    out_type=jax.ShapeDtypeStruct((POSE_DIM, BATCH), jnp.float32),
    scratch_types=[
        pltpu.VMEM((_B_PER_W,), jnp.int32),
        pltpu.VMEM((_B_PER_W,), jnp.int32),
        pltpu.VMEM((_B_PER_W,), jnp.int32),
        pltpu.VMEM((POSE_DIM * _B_PER_W, 8), jnp.float32),
        pltpu.VMEM((POSE_DIM, _B_PER_W), jnp.float32),
    ] + [pltpu.SemaphoreType.DMA] * POSE_DIM,
    compiler_params=pltpu.CompilerParams(use_tc_tiling_on_sc=False,
                                         needs_layout_passes=False,
                                         skip_device_barrier=True,
                                         disable_bounds_checks=True,
                                         disable_semaphore_checks=True),
)
def _gather_sc(tab_hbm, idx_hbm, out_hbm, idx_v, low_v, blk_v, win_v,
               rows_v, *sems):
    wid = lax.axis_index("s") * _NUM_CORES + lax.axis_index("c")
    base = wid * _B_PER_W
    pltpu.sync_copy(idx_hbm.at[pl.ds(base, _B_PER_W)], idx_v)

    iota = lax.iota(jnp.int32, _LANES)

    for t in range(_B_PER_W // _LANES):
        s = t * _LANES
        v = idx_v[pl.ds(s, _LANES)]
        low_v[pl.ds(s, _LANES)] = lax.bitwise_and(v, 7)
        blk_v[pl.ds(s, _LANES)] = lax.shift_right_logical(v, 3)

    copies = [
        pltpu.async_copy(
            tab_hbm.at[pl.ds(j * _COL_BLKS, _COL_BLKS)].at[blk_v],
            win_v.at[pl.ds(j * _B_PER_W, _B_PER_W)], sems[j])
        for j in range(POSE_DIM)
    ]

    out_copies = []
    for j in range(POSE_DIM):
        copies[j].wait()

        for t in range(_B_PER_W // _LANES):
            s = t * _LANES
            lo = low_v[pl.ds(s, _LANES)]
            rows16 = (j * _B_PER_W + s) + iota
            rows_v[j, pl.ds(s, _LANES)] = (
                plsc.load_gather(win_v, [rows16, lo]))
        out_copies.append(pltpu.async_copy(
            rows_v.at[j], out_hbm.at[j, pl.ds(base, _B_PER_W)], sems[j]))

    for cp in out_copies:
        cp.wait()


def kernel(d9, i):
    tab = d9.T.reshape(NUM_POSES * POSE_DIM // 8, 8)
    out = _gather_sc(tab, i.astype(jnp.int32))
    return out.T
